# baseline (speedup 1.0000x reference)
import jax
import jax.numpy as jnp
from jax.experimental import pallas as pl


def _embed_kernel(dt_ref, table_ref, out_ref):
    c, e, b = out_ref.shape
    d3 = jnp.broadcast_to(dt_ref[...][:, None, :], (c, e, b))
    t = table_ref[...]
    t0 = t[0][:, None]
    t1 = t[1][:, None]
    t2 = t[2][:, None]
    t3 = t[3][:, None]
    out_bf = jnp.where(
        d3 < 2.0,
        jnp.where(d3 == 0.0, t0, t1),
        jnp.where(d3 == 2.0, t2, t3),
    )
    out_ref[...] = out_bf.astype(jnp.float32)


def kernel(date, table):
    n, c = date.shape
    e = table.shape[1]
    dt = jnp.swapaxes(date, 0, 1).astype(jnp.bfloat16)
    tb = table.astype(jnp.bfloat16)
    block = 1024
    grid = (n // block,)
    out_t = pl.pallas_call(
        _embed_kernel,
        grid=grid,
        in_specs=[
            pl.BlockSpec((c, block), lambda i: (0, i)),
            pl.BlockSpec((4, e), lambda i: (0, 0)),
        ],
        out_specs=pl.BlockSpec((c, e, block), lambda i: (0, 0, i)),
        out_shape=jax.ShapeDtypeStruct((c, e, n), jnp.float32),
    )(dt, tb)
    return jnp.transpose(out_t, (2, 0, 1))


# speedup vs baseline: 1.0900x; 1.0900x over previous
import jax
import jax.numpy as jnp
from jax.experimental import pallas as pl


def _embed_kernel(dt_ref, table_ref, out_ref):
    c, e, b = out_ref.shape
    db = dt_ref[...].astype(jnp.bfloat16)
    d3 = jnp.broadcast_to(db[:, None, :], (c, e, b))
    t = table_ref[...]
    t0 = t[0][:, None]
    t1 = t[1][:, None]
    t2 = t[2][:, None]
    t3 = t[3][:, None]
    out_bf = jnp.where(
        d3 < 2.0,
        jnp.where(d3 == 0.0, t0, t1),
        jnp.where(d3 == 2.0, t2, t3),
    )
    out_ref[...] = out_bf.astype(jnp.float32)


def kernel(date, table):
    n, c = date.shape
    e = table.shape[1]
    dt = jnp.swapaxes(date, 0, 1)
    tb = table.astype(jnp.bfloat16)
    block = 1024
    grid = (n // block,)
    out_t = pl.pallas_call(
        _embed_kernel,
        grid=grid,
        in_specs=[
            pl.BlockSpec((c, block), lambda i: (0, i)),
            pl.BlockSpec((4, e), lambda i: (0, 0)),
        ],
        out_specs=pl.BlockSpec((c, e, block), lambda i: (0, 0, i)),
        out_shape=jax.ShapeDtypeStruct((c, e, n), jnp.float32),
    )(dt, tb)
    return jnp.transpose(out_t, (2, 0, 1))
